# baseline (speedup 1.0000x reference)
import jax
import jax.numpy as jnp
from jax.experimental import pallas as pl
from jax.experimental.pallas import tpu as pltpu


def _body(t_ref, w_ref, o_ref):
    s = jax.nn.sigmoid(t_ref[0, 0])
    w = w_ref[...]
    o_ref[...] = jnp.sign(w) * jnp.maximum(jnp.abs(w) - s, 0.0)


def kernel(weight, threshold, alpha):
    R, C = weight.shape
    BR = 256
    return pl.pallas_call(
        _body,
        grid=(R // BR,),
        in_specs=[
            pl.BlockSpec(memory_space=pltpu.SMEM),
            pl.BlockSpec((BR, C), lambda i: (i, 0)),
        ],
        out_specs=pl.BlockSpec((BR, C), lambda i: (i, 0)),
        out_shape=jax.ShapeDtypeStruct((R, C), jnp.float32),
    )(threshold, weight)
